# baseline (device time: 25501 ns/iter reference)
import jax
import jax.numpy as jnp
from jax import lax
from jax.experimental import pallas as pl
from jax.experimental.pallas import tpu as pltpu

N_DEV = 4
B, SQ, SKV = 2, 256, 256
HQ_LOC, DH = 4, 64
DM = 512
BLK = 64
RC = SQ // 2
NR = SQ // RC
NCH = B * NR


def kernel(x, Wq, K_ext, V_ext, Wo):
    def body(x_ref, wq_ref, k_hbm, v_hbm, wo_ref, out_ref,
             k_st, v_st, k_ref, v_ref, send0_ref, send1_ref, recv_ref,
             load_sems, ds_sems, send_sems, recv_sems):
        my_p = lax.axis_index("i")
        partner1 = my_p ^ 1
        partner2 = 3 - my_p

        stage_loads = {}
        for b in range(B):
            kl = pltpu.make_async_copy(
                k_hbm.at[b, :, pl.ds(my_p * HQ_LOC, HQ_LOC), :],
                k_st.at[b], load_sems.at[b])
            vl = pltpu.make_async_copy(
                v_hbm.at[b, :, pl.ds(my_p * HQ_LOC, HQ_LOC), :],
                v_st.at[b], load_sems.at[B + b])
            kl.start()
            vl.start()
            stage_loads[b] = (kl, vl)
        loads_pending = {0: True, 1: True}

        def ensure_kv(b):
            if not loads_pending[b]:
                return
            loads_pending[b] = False
            for cp in stage_loads[b]:
                cp.wait()
            destrides = []
            for h in range(HQ_LOC):
                for j, (src, dst) in enumerate(((k_st, k_ref), (v_st, v_ref))):
                    cp = pltpu.make_async_copy(
                        src.at[b, :, h, :], dst.at[b, h],
                        ds_sems.at[b * 2 * HQ_LOC + h * 2 + j])
                    cp.start()
                    destrides.append(cp)
            for cp in destrides:
                cp.wait()

        barrier = pltpu.get_barrier_semaphore()
        for nbr in (partner1, partner2):
            pl.semaphore_signal(
                barrier, inc=1,
                device_id=(nbr,), device_id_type=pl.DeviceIdType.MESH,
            )

        order = [(b, r) for r in range(NR) for b in range(B)]

        def compute_chunk(b, r):
            ci = b * NR + r
            kn = (r + 1) * RC
            rows = slice(r * RC, (r + 1) * RC)
            qb = lax.broadcasted_iota(jnp.int32, (RC, kn), 0) // BLK + (
                r * RC // BLK)
            kb = lax.broadcasted_iota(jnp.int32, (RC, kn), 1) // BLK
            mask = kb <= qb
            q_all = jnp.dot(x_ref[b, rows], wq_ref[...],
                            preferred_element_type=jnp.float32) * 0.125
            ensure_kv(b)
            acc = jnp.zeros((RC, DM), jnp.float32)
            for h in range(HQ_LOC):
                q = q_all[:, h * DH:(h + 1) * DH]
                k = k_ref[b, h, :kn]
                v = v_ref[b, h, :kn]
                s = lax.dot_general(
                    q, k, (((1,), (1,)), ((), ())),
                    preferred_element_type=jnp.float32)
                w = jnp.exp(jnp.where(mask, s, -1e9))
                w = w / jnp.sum(w, axis=-1, keepdims=True)
                ctx = jnp.dot(w, v, preferred_element_type=jnp.float32)
                acc = acc + jnp.dot(ctx, wo_ref[h * DH:(h + 1) * DH, :],
                                    preferred_element_type=jnp.float32)
            out_ref[b, rows] = acc
            send0_ref[ci] = acc.astype(jnp.bfloat16)

        def start_ex(stage, ci, partner, src_ref):
            idx = stage * NCH + ci
            rdma = pltpu.make_async_remote_copy(
                src_ref=src_ref.at[ci],
                dst_ref=recv_ref.at[idx],
                send_sem=send_sems.at[idx],
                recv_sem=recv_sems.at[idx],
                device_id=(partner,),
                device_id_type=pl.DeviceIdType.MESH,
            )
            rdma.start()
            return rdma

        s0 = {}
        s1 = {}

        def finish0(b, r):
            ci = b * NR + r
            s0[ci].wait()
            rows = slice(r * RC, (r + 1) * RC)
            tmp = out_ref[b, rows] + recv_ref[ci].astype(jnp.float32)
            out_ref[b, rows] = tmp
            send1_ref[ci] = tmp.astype(jnp.bfloat16)
            s1[ci] = start_ex(1, ci, partner2 if b == 0 else partner1,
                              send1_ref)

        LAG = 2
        for i, (b, r) in enumerate(order):
            compute_chunk(b, r)
            if i == 0:
                pl.semaphore_wait(barrier, 2)
            s0[b * NR + r] = start_ex(0, b * NR + r,
                                      partner1 if b == 0 else partner2,
                                      send0_ref)
            if i >= LAG:
                finish0(*order[i - LAG])
        for (b, r) in order[-LAG:]:
            finish0(b, r)

        for (b, r) in order:
            ci = b * NR + r
            s1[ci].wait()
            rows = slice(r * RC, (r + 1) * RC)
            out_ref[b, rows] = (
                out_ref[b, rows] + recv_ref[NCH + ci].astype(jnp.float32))

    return pl.pallas_call(
        body,
        out_shape=jax.ShapeDtypeStruct((B, SQ, DM), jnp.float32),
        in_specs=[
            pl.BlockSpec(memory_space=pltpu.VMEM),
            pl.BlockSpec(memory_space=pltpu.VMEM),
            pl.BlockSpec(memory_space=pl.ANY),
            pl.BlockSpec(memory_space=pl.ANY),
            pl.BlockSpec(memory_space=pltpu.VMEM),
        ],
        out_specs=pl.BlockSpec(memory_space=pltpu.VMEM),
        scratch_shapes=[
            pltpu.VMEM((B, SKV, HQ_LOC, DH), jnp.float32),
            pltpu.VMEM((B, SKV, HQ_LOC, DH), jnp.float32),
            pltpu.VMEM((B, HQ_LOC, SKV, DH), jnp.float32),
            pltpu.VMEM((B, HQ_LOC, SKV, DH), jnp.float32),
            pltpu.VMEM((NCH, RC, DM), jnp.bfloat16),
            pltpu.VMEM((NCH, RC, DM), jnp.bfloat16),
            pltpu.VMEM((2 * NCH, RC, DM), jnp.bfloat16),
            pltpu.SemaphoreType.DMA((2 * B,)),
            pltpu.SemaphoreType.DMA((2 * B * HQ_LOC,)),
            pltpu.SemaphoreType.DMA((2 * NCH,)),
            pltpu.SemaphoreType.DMA((2 * NCH,)),
        ],
        compiler_params=pltpu.CompilerParams(collective_id=0),
    )(x, Wq, K_ext, V_ext, Wo)


# device time: 18535 ns/iter; 1.3758x vs baseline; 1.3758x over previous
import jax
import jax.numpy as jnp
from jax import lax
from jax.experimental import pallas as pl
from jax.experimental.pallas import tpu as pltpu

N_DEV = 4
B, SQ, SKV = 2, 256, 256
HQ_LOC, DH = 4, 64
DM = 512
BLK = 64
RC = SQ // 2
NR = SQ // RC
NCH = B * NR


def kernel(x, Wq, K_ext, V_ext, Wo):
    p = lax.axis_index("i")
    K_loc = lax.dynamic_slice_in_dim(K_ext, p * HQ_LOC, HQ_LOC, axis=2)
    V_loc = lax.dynamic_slice_in_dim(V_ext, p * HQ_LOC, HQ_LOC, axis=2)
    K_loc = jnp.transpose(K_loc, (0, 2, 1, 3)).astype(jnp.bfloat16)
    V_loc = jnp.transpose(V_loc, (0, 2, 1, 3)).astype(jnp.bfloat16)

    def body(x_ref, wq_ref, k_ref, v_ref, wo_ref, out_ref,
             send0_ref, send1_ref, recv_ref, send_sems, recv_sems):
        my_p = lax.axis_index("i")
        partner1 = my_p ^ 1
        partner2 = 3 - my_p

        barrier = pltpu.get_barrier_semaphore()
        for nbr in (partner1, partner2):
            pl.semaphore_signal(
                barrier, inc=1,
                device_id=(nbr,), device_id_type=pl.DeviceIdType.MESH,
            )

        order = [(b, r) for r in range(NR) for b in range(B)]

        def compute_chunk(b, r):
            ci = b * NR + r
            kn = (r + 1) * RC
            rows = slice(r * RC, (r + 1) * RC)
            qb = lax.broadcasted_iota(jnp.int32, (RC, kn), 0) // BLK + (
                r * RC // BLK)
            kb = lax.broadcasted_iota(jnp.int32, (RC, kn), 1) // BLK
            mask = kb <= qb
            q_all = (jnp.dot(x_ref[b, rows], wq_ref[...],
                             preferred_element_type=jnp.float32)
                     * 0.125).astype(jnp.bfloat16)
            acc = jnp.zeros((RC, DM), jnp.float32)
            for h in range(HQ_LOC):
                q = q_all[:, h * DH:(h + 1) * DH]
                k = k_ref[b, h, :kn]
                v = v_ref[b, h, :kn]
                s = lax.dot_general(
                    q, k, (((1,), (1,)), ((), ())),
                    preferred_element_type=jnp.float32)
                w = jnp.exp(jnp.where(mask, s, -1e9))
                w = (w / jnp.sum(w, axis=-1, keepdims=True)).astype(
                    jnp.bfloat16)
                ctx = jnp.dot(w, v, preferred_element_type=jnp.float32)
                acc = acc + jnp.dot(ctx, wo_ref[h * DH:(h + 1) * DH, :],
                                    preferred_element_type=jnp.float32)
            out_ref[b, rows] = acc
            send0_ref[ci] = acc.astype(jnp.bfloat16)

        def start_ex(stage, ci, partner, src_ref):
            idx = stage * NCH + ci
            rdma = pltpu.make_async_remote_copy(
                src_ref=src_ref.at[ci],
                dst_ref=recv_ref.at[idx],
                send_sem=send_sems.at[idx],
                recv_sem=recv_sems.at[idx],
                device_id=(partner,),
                device_id_type=pl.DeviceIdType.MESH,
            )
            rdma.start()
            return rdma

        s0 = {}
        s1 = {}

        def finish0(b, r):
            ci = b * NR + r
            s0[ci].wait()
            rows = slice(r * RC, (r + 1) * RC)
            tmp = out_ref[b, rows] + recv_ref[ci].astype(jnp.float32)
            out_ref[b, rows] = tmp
            send1_ref[ci] = tmp.astype(jnp.bfloat16)
            s1[ci] = start_ex(1, ci, partner2 if b == 0 else partner1,
                              send1_ref)

        LAG = 2
        for i, (b, r) in enumerate(order):
            compute_chunk(b, r)
            if i == 0:
                pl.semaphore_wait(barrier, 2)
            s0[b * NR + r] = start_ex(0, b * NR + r,
                                      partner1 if b == 0 else partner2,
                                      send0_ref)
            if i >= LAG:
                finish0(*order[i - LAG])
        for (b, r) in order[-LAG:]:
            finish0(b, r)

        for (b, r) in order:
            ci = b * NR + r
            s1[ci].wait()
            rows = slice(r * RC, (r + 1) * RC)
            out_ref[b, rows] = (
                out_ref[b, rows] + recv_ref[NCH + ci].astype(jnp.float32))

    return pl.pallas_call(
        body,
        out_shape=jax.ShapeDtypeStruct((B, SQ, DM), jnp.float32),
        in_specs=[pl.BlockSpec(memory_space=pltpu.VMEM)] * 5,
        out_specs=pl.BlockSpec(memory_space=pltpu.VMEM),
        scratch_shapes=[
            pltpu.VMEM((NCH, RC, DM), jnp.bfloat16),
            pltpu.VMEM((NCH, RC, DM), jnp.bfloat16),
            pltpu.VMEM((2 * NCH, RC, DM), jnp.bfloat16),
            pltpu.SemaphoreType.DMA((2 * NCH,)),
            pltpu.SemaphoreType.DMA((2 * NCH,)),
        ],
        compiler_params=pltpu.CompilerParams(collective_id=0),
    )(x, Wq, K_loc, V_loc, Wo)


# device time: 15121 ns/iter; 1.6865x vs baseline; 1.2258x over previous
import jax
import jax.numpy as jnp
from jax import lax
from jax.experimental import pallas as pl
from jax.experimental.pallas import tpu as pltpu

N_DEV = 4
B, SQ, SKV = 2, 256, 256
HQ_LOC, DH = 4, 64
DM = 512
BLK = 64
RC = SQ // 2
NR = SQ // RC
SC = 64
NS = SQ // SC
NSC = B * NS


def kernel(x, Wq, K_ext, V_ext, Wo):
    p = lax.axis_index("i")
    K_loc = lax.dynamic_slice_in_dim(K_ext, p * HQ_LOC, HQ_LOC, axis=2)
    V_loc = lax.dynamic_slice_in_dim(V_ext, p * HQ_LOC, HQ_LOC, axis=2)
    K_loc = jnp.transpose(K_loc, (0, 2, 1, 3)).astype(jnp.bfloat16)
    V_loc = jnp.transpose(V_loc, (0, 2, 1, 3)).astype(jnp.bfloat16)

    def body(x_ref, wq_ref, k_ref, v_ref, wo_ref, out_ref,
             recv_ref, send_sems, recv_sems):
        my_p = lax.axis_index("i")
        partner1 = my_p ^ 1
        partner2 = 3 - my_p

        barrier = pltpu.get_barrier_semaphore()
        for nbr in (partner1, partner2):
            pl.semaphore_signal(
                barrier, inc=1,
                device_id=(nbr,), device_id_type=pl.DeviceIdType.MESH,
            )

        order = [(b, r) for r in range(NR) for b in range(B)]

        def compute_chunk(b, r):
            ci = b * NR + r
            kn = (r + 1) * RC
            rows = slice(r * RC, (r + 1) * RC)
            qb = lax.broadcasted_iota(jnp.int32, (RC, kn), 0) // BLK + (
                r * RC // BLK)
            kb = lax.broadcasted_iota(jnp.int32, (RC, kn), 1) // BLK
            mask = kb <= qb
            q_all = (jnp.dot(x_ref[b, rows], wq_ref[...],
                             preferred_element_type=jnp.float32)
                     * 0.125).astype(jnp.bfloat16)
            ctxs = []
            for h in range(HQ_LOC):
                q = q_all[:, h * DH:(h + 1) * DH]
                k = k_ref[b, h, :kn]
                v = v_ref[b, h, :kn]
                s = lax.dot_general(
                    q, k, (((1,), (1,)), ((), ())),
                    preferred_element_type=jnp.float32)
                w = jnp.exp(jnp.where(mask, s, -1e9)).astype(jnp.bfloat16)
                denom = jnp.sum(w.astype(jnp.float32), axis=-1,
                                keepdims=True)
                ctx = jnp.dot(w, v, preferred_element_type=jnp.float32)
                ctxs.append((ctx / denom).astype(jnp.bfloat16))
            ctx_all = jnp.concatenate(ctxs, axis=-1)
            acc = jnp.dot(ctx_all, wo_ref[...],
                          preferred_element_type=jnp.float32)
            out_ref[b, rows] = acc.astype(jnp.bfloat16)

        def srows(j):
            return slice(j * SC, (j + 1) * SC)

        def start_ex(stage, b, j):
            if stage == 0:
                partner = partner1 if b == 0 else partner2
            else:
                partner = partner2 if b == 0 else partner1
            idx = stage * NSC + b * NS + j
            rdma = pltpu.make_async_remote_copy(
                src_ref=out_ref.at[b, j * SC:(j + 1) * SC],
                dst_ref=recv_ref.at[idx],
                send_sem=send_sems.at[idx],
                recv_sem=recv_sems.at[idx],
                device_id=(partner,),
                device_id_type=pl.DeviceIdType.MESH,
            )
            rdma.start()
            return rdma

        s0 = {}
        s1 = {}

        def finish0(b, j):
            si = b * NS + j
            s0[si].wait()
            out_ref[b, srows(j)] = out_ref[b, srows(j)] + recv_ref[si]
            s1[si] = start_ex(1, b, j)

        LAG = 4
        for i, (b, r) in enumerate(order):
            compute_chunk(b, r)
            if i == 0:
                pl.semaphore_wait(barrier, 2)
            for j in range(r * RC // SC, (r + 1) * RC // SC):
                s0[b * NS + j] = start_ex(0, b, j)
            if i >= LAG:
                bb, rr = order[i - LAG]
                for j in range(rr * RC // SC, (rr + 1) * RC // SC):
                    finish0(bb, j)
        for (b, r) in order[-LAG:]:
            for j in range(r * RC // SC, (r + 1) * RC // SC):
                finish0(b, j)

        for (b, r) in order:
            for j in range(r * RC // SC, (r + 1) * RC // SC):
                si = b * NS + j
                s1[si].wait()
                out_ref[b, srows(j)] = (
                    out_ref[b, srows(j)] + recv_ref[NSC + si])

    return pl.pallas_call(
        body,
        out_shape=jax.ShapeDtypeStruct((B, SQ, DM), jnp.bfloat16),
        in_specs=[pl.BlockSpec(memory_space=pltpu.VMEM)] * 5,
        out_specs=pl.BlockSpec(memory_space=pltpu.VMEM),
        scratch_shapes=[
            pltpu.VMEM((2 * NSC, SC, DM), jnp.bfloat16),
            pltpu.SemaphoreType.DMA((2 * NSC,)),
            pltpu.SemaphoreType.DMA((2 * NSC,)),
        ],
        compiler_params=pltpu.CompilerParams(collective_id=0),
    )(x, Wq, K_loc, V_loc, Wo)
